# baseline (speedup 1.0000x reference)
import functools

import jax
import jax.numpy as jnp
from jax import lax
from jax.experimental import pallas as pl
from jax.experimental.pallas import tpu as pltpu, tpu_sc as plsc


def _gather_kernel(B, D):
    info = plsc.get_sparse_core_info()
    NC, NS, L = info.num_cores, info.num_subcores, info.num_lanes
    NW = NC * NS
    assert B % NW == 0 and D % L == 0
    b_per_w = B // NW

    mesh = plsc.VectorSubcoreMesh(core_axis_name="c", subcore_axis_name="s")

    @functools.partial(
        pl.kernel,
        mesh=mesh,
        out_type=jax.ShapeDtypeStruct((B, D), jnp.float32),
        scratch_types=[
            pltpu.VMEM((b_per_w,), jnp.int32),
            pltpu.VMEM((b_per_w,), jnp.int32),
            pltpu.VMEM((b_per_w // 2, 2 * D), jnp.float32),
            pltpu.VMEM((b_per_w, D), jnp.float32),
            pltpu.SemaphoreType.DMA,
        ],
        compiler_params=pltpu.CompilerParams(
            needs_layout_passes=False, use_tc_tiling_on_sc=False),
    )
    def k(idx_hbm, table2_hbm, out_hbm, idx_v, idx2_v, pairs_v, out_v, sem):
        wid = lax.axis_index("s") * NC + lax.axis_index("c")
        base = wid * b_per_w
        pltpu.sync_copy(idx_hbm.at[pl.ds(base, b_per_w)], idx_v)

        lane = lax.iota(jnp.int32, L)

        def halve_body(g):
            vals = plsc.load_gather(idx_v, [g * L + lane])
            plsc.store_scatter(idx2_v, [g * L + lane], vals >> 1)

        pl.loop(0, b_per_w // L)(halve_body)

        half_n = b_per_w // 2
        for ch in range(2):
            cp = pltpu.async_copy(
                table2_hbm.at[idx2_v.at[pl.ds(ch * half_n, half_n)]],
                pairs_v, sem)
            cp.wait()

            def select_body(j):
                b = ch * half_n + j
                b_vec = jnp.full((L,), b, jnp.int32)
                j_vec = jnp.full((L,), j, jnp.int32)
                half = (plsc.load_gather(idx_v, [b_vec]) & 1) * D
                for cg in range(D // L):
                    vals = plsc.load_gather(pairs_v,
                                            [j_vec, half + cg * L + lane])
                    plsc.store_scatter(out_v, [b_vec, cg * L + lane], vals)

            pl.loop(0, half_n)(select_body)

        pltpu.sync_copy(out_v, out_hbm.at[pl.ds(base, b_per_w)])

    return k


def kernel(indices, embed):
    (B,) = indices.shape
    V, D = embed.shape
    table2 = jnp.reshape(embed, (V // 2, 2 * D))
    return _gather_kernel(B, D)(indices.astype(jnp.int32), table2)


# speedup vs baseline: 1.5616x; 1.5616x over previous
import functools

import jax
import jax.numpy as jnp
from jax import lax
from jax.experimental import pallas as pl
from jax.experimental.pallas import tpu as pltpu, tpu_sc as plsc


def _gather_kernel(B, D):
    info = plsc.get_sparse_core_info()
    NC, NS, L = info.num_cores, info.num_subcores, info.num_lanes
    NW = NC * NS
    assert B % NW == 0 and D % L == 0
    b_per_w = B // NW
    GROUP = 16
    assert b_per_w % GROUP == 0

    mesh = plsc.VectorSubcoreMesh(core_axis_name="c", subcore_axis_name="s")

    @functools.partial(
        pl.kernel,
        mesh=mesh,
        out_type=jax.ShapeDtypeStruct((B, D), jnp.float32),
        scratch_types=[
            pltpu.VMEM((b_per_w,), jnp.int32),
            pltpu.VMEM((GROUP * 8, D), jnp.float32),
            pltpu.VMEM((b_per_w, D), jnp.float32),
            pltpu.SemaphoreType.DMA,
        ],
        compiler_params=pltpu.CompilerParams(needs_layout_passes=False),
    )
    def k(idx_hbm, table_hbm, out_hbm, idx_v, stage_v, out_v, sem):
        wid = lax.axis_index("s") * NC + lax.axis_index("c")
        base = wid * b_per_w
        pltpu.sync_copy(idx_hbm.at[pl.ds(base, b_per_w)], idx_v)

        lane = lax.iota(jnp.int32, L)

        def group_body(g):
            vgrp = idx_v[pl.ds(g * GROUP, GROUP)]
            copies = []
            for j in range(GROUP):
                off = pl.multiple_of((vgrp[j] >> 3) * 8, 8)
                copies.append(
                    pltpu.async_copy(
                        table_hbm.at[pl.ds(off, 8), :],
                        stage_v.at[pl.ds(j * 8, 8), :],
                        sem,
                    )
                )
            for cp in copies:
                cp.wait()
            for j in range(GROUP):
                i = g * GROUP + j
                src_row = jnp.full((L,), j * 8, jnp.int32) + (vgrp[j] & 7)
                i_vec = jnp.full((L,), i, jnp.int32)
                for cg in range(D // L):
                    vals = plsc.load_gather(stage_v,
                                            [src_row, cg * L + lane])
                    plsc.store_scatter(out_v, [i_vec, cg * L + lane], vals)

        pl.loop(0, b_per_w // GROUP)(group_body)
        pltpu.sync_copy(out_v, out_hbm.at[pl.ds(base, b_per_w)])

    return k


def kernel(indices, embed):
    (B,) = indices.shape
    _, D = embed.shape
    return _gather_kernel(B, D)(indices.astype(jnp.int32), embed)


# speedup vs baseline: 1.6291x; 1.0432x over previous
import functools

import jax
import jax.numpy as jnp
from jax import lax
from jax.experimental import pallas as pl
from jax.experimental.pallas import tpu as pltpu, tpu_sc as plsc


def _gather_kernel(B, D):
    info = plsc.get_sparse_core_info()
    NC, NS, L = info.num_cores, info.num_subcores, info.num_lanes
    NW = NC * NS
    assert B % NW == 0 and D % L == 0
    b_per_w = B // NW
    GROUP = 16
    assert b_per_w % GROUP == 0

    mesh = plsc.VectorSubcoreMesh(core_axis_name="c", subcore_axis_name="s")

    @functools.partial(
        pl.kernel,
        mesh=mesh,
        out_type=jax.ShapeDtypeStruct((B, D), jnp.float32),
        scratch_types=[
            pltpu.VMEM((b_per_w,), jnp.int32),
            pltpu.VMEM((GROUP * 8, D), jnp.float32),
            pltpu.VMEM((GROUP * 8, D), jnp.float32),
            pltpu.VMEM((b_per_w, D), jnp.float32),
            pltpu.SemaphoreType.DMA,
            pltpu.SemaphoreType.DMA,
        ],
        compiler_params=pltpu.CompilerParams(needs_layout_passes=False),
    )
    def k(idx_hbm, table_hbm, out_hbm, idx_v, stage0, stage1, out_v,
          sem0, sem1):
        wid = lax.axis_index("s") * NC + lax.axis_index("c")
        base = wid * b_per_w
        pltpu.sync_copy(idx_hbm.at[pl.ds(base, b_per_w)], idx_v)

        lane = lax.iota(jnp.int32, L)
        n_groups = b_per_w // GROUP

        def fire(g, buf, sem):
            vgrp = idx_v[pl.ds(g * GROUP, GROUP)]
            for j in range(GROUP):
                off = pl.multiple_of((vgrp[j] >> 3) * 8, 8)
                pltpu.async_copy(
                    table_hbm.at[pl.ds(off, 8), :],
                    buf.at[pl.ds(j * 8, 8), :],
                    sem,
                )

        def drain_select(g, buf, sem):
            pltpu.make_async_copy(
                table_hbm.at[pl.ds(0, GROUP * 8), :], buf, sem).wait()
            vgrp = idx_v[pl.ds(g * GROUP, GROUP)]
            for j in range(GROUP):
                i = g * GROUP + j
                src_row = jnp.full((L,), j * 8, jnp.int32) + (vgrp[j] & 7)
                i_vec = jnp.full((L,), i, jnp.int32)
                for cg in range(D // L):
                    vals = plsc.load_gather(buf, [src_row, cg * L + lane])
                    plsc.store_scatter(out_v, [i_vec, cg * L + lane], vals)

        fire(0, stage0, sem0)

        def pipe_body(h):
            g = h * 2
            fire(g + 1, stage1, sem1)
            drain_select(g, stage0, sem0)
            fire(g + 2, stage0, sem0)
            drain_select(g + 1, stage1, sem1)

        pl.loop(0, n_groups // 2 - 1)(pipe_body)
        g_last = n_groups - 2
        fire(g_last + 1, stage1, sem1)
        drain_select(g_last, stage0, sem0)
        drain_select(g_last + 1, stage1, sem1)
        pltpu.sync_copy(out_v, out_hbm.at[pl.ds(base, b_per_w)])

    return k


def kernel(indices, embed):
    (B,) = indices.shape
    _, D = embed.shape
    return _gather_kernel(B, D)(indices.astype(jnp.int32), embed)
